# baseline (speedup 1.0000x reference)
import functools

import jax
import jax.numpy as jnp
from jax import lax
from jax.experimental import pallas as pl
from jax.experimental.pallas import tpu as pltpu
from jax.experimental.pallas import tpu_sc as plsc

VOC_SIZE = 1000000
EMB_DIM = 32
BATCH = 16384
HIST_LEN = 200
RW = 128
XROWS = BATCH * HIST_LEN // RW

_info = plsc.get_sparse_core_info()
NC = _info.num_cores
NS = _info.num_subcores
NW = NC * NS
ROWS_PER_W = BATCH // NW
XR_PER_W = XROWS // NW
PHASE_XR = 200
NPHASE = XR_PER_W // PHASE_XR
PHASE_BROWS = PHASE_XR * RW // HIST_LEN
NBUF = 4
NGROUP = PHASE_XR // NBUF
NGS = RW // 8


def _hsum16(vec, scratch_ref):
    lane = lax.iota(jnp.int32, 16)
    for sh in (8, 4, 2, 1):
        scratch_ref[...] = vec
        vec = vec + plsc.load_gather(scratch_ref, [lane ^ sh])
    return vec


def _body(x_hbm, w_hbm, out_hbm, idx_v, bufs, gsb, acc_v, cnt_v, out_v,
          w0_v, hs_v, sems):
    wid = lax.axis_index("s") * NC + lax.axis_index("c")
    xr_base = wid * XR_PER_W

    pltpu.sync_copy(w_hbm.at[pl.ds(0, 8)], w0_v)
    w0a = w0_v[0, pl.ds(0, 16)]
    w0b = w0_v[0, pl.ds(16, 16)]

    zero = jnp.zeros((16,), jnp.float32)
    izero = jnp.zeros((16,), jnp.int32)
    ione = jnp.ones((16,), jnp.int32)
    lane = lax.iota(jnp.int32, 16)

    def fire(r, b):
        pltpu.async_copy(w_hbm.at[idx_v.at[r]], bufs[b], sems[b])

    def drain(r, b):
        pltpu.make_async_copy(w_hbm.at[idx_v.at[r]], bufs[b], sems[b]).wait()

    def phase(p, pcarry):
        pltpu.sync_copy(
            x_hbm.at[pl.ds(xr_base + p * PHASE_XR, PHASE_XR)], idx_v
        )
        acc_v[0, pl.ds(0, 16)] = zero
        acc_v[1, pl.ds(0, 16)] = zero
        cnt_v[...] = izero
        for b in range(NBUF):
            fire(b, b)

        def group(g, carry):
            j, bpos = carry
            more = g < NGROUP - 1
            for b in range(NBUF):
                r = g * NBUF + b
                fs = (g * NBUF + b) * RW
                sp = bpos - fs
                drain(r, b)
                rv = bufs[b]
                gs = gsb[b]
                t0 = zero
                t1 = zero
                for gi in range(NGS):
                    s0 = zero
                    s1 = zero
                    for s in range(8 * gi, 8 * gi + 8):
                        s0 = s0 + rv[s, pl.ds(0, 16)]
                        s1 = s1 + rv[s, pl.ds(16, 16)]
                    gs[gi, pl.ds(0, 16)] = s0
                    gs[gi, pl.ds(16, 16)] = s1
                    t0 = t0 + s0
                    t1 = t1 + s1
                ct = izero
                for o in range(0, RW, 16):
                    v = idx_v[r, pl.ds(o, 16)]
                    ct = ct + jnp.where(v == 0, ione, izero)

                @pl.when(more)
                def _(r=r, b=b):
                    fire(r + NBUF, b)

                has_b = sp <= RW

                @pl.when(has_b)
                def _(sp=sp, t0=t0, t1=t1, ct=ct, j=j, r=r, gs=gs):
                    nsp = sp >> 3
                    def suf_add(k, sacc):
                        return (
                            sacc[0] + gs[k, pl.ds(0, 16)],
                            sacc[1] + gs[k, pl.ds(16, 16)],
                        )

                    suf0, suf1 = lax.fori_loop(nsp, NGS, suf_add, (zero, zero))
                    spv = jnp.full((16,), sp, jnp.int32)
                    csuf = izero
                    for o in range(0, RW, 16):
                        v = idx_v[r, pl.ds(o, 16)]
                        csuf = csuf + jnp.where(
                            jnp.logical_and(v == 0, (o + lane) >= spv),
                            ione,
                            izero,
                        )
                    fin0 = acc_v[0, pl.ds(0, 16)] + t0 - suf0
                    fin1 = acc_v[1, pl.ds(0, 16)] + t1 - suf1
                    finc = cnt_v[...] + ct - csuf
                    nz = _hsum16(finc, hs_v)
                    nzf = nz.astype(jnp.float32)
                    cntf = (HIST_LEN - nz).astype(jnp.float32)
                    out_v[j, pl.ds(0, 16)] = (fin0 - nzf * w0a) / cntf
                    out_v[j, pl.ds(16, 16)] = (fin1 - nzf * w0b) / cntf
                    acc_v[0, pl.ds(0, 16)] = suf0
                    acc_v[1, pl.ds(0, 16)] = suf1
                    cnt_v[...] = csuf

                @pl.when(jnp.logical_not(has_b))
                def _(t0=t0, t1=t1, ct=ct):
                    acc_v[0, pl.ds(0, 16)] = acc_v[0, pl.ds(0, 16)] + t0
                    acc_v[1, pl.ds(0, 16)] = acc_v[1, pl.ds(0, 16)] + t1
                    cnt_v[...] = cnt_v[...] + ct

                hbi = jnp.where(has_b, 1, 0)
                j = j + hbi
                bpos = bpos + HIST_LEN * hbi
            return (j, bpos)

        lax.fori_loop(
            0, NGROUP, group, (p * PHASE_BROWS, jnp.int32(HIST_LEN))
        )
        return pcarry

    lax.fori_loop(0, NPHASE, phase, 0)

    pltpu.sync_copy(out_v, out_hbm.at[pl.ds(wid * ROWS_PER_W, ROWS_PER_W)])


@functools.partial(jax.jit, donate_argnums=())
def kernel(X, W):
    xf = X.astype(jnp.int32).reshape(XROWS, RW)
    mesh = plsc.VectorSubcoreMesh(core_axis_name="c", subcore_axis_name="s")
    k = pl.kernel(
        _body,
        mesh=mesh,
        out_type=jax.ShapeDtypeStruct((BATCH, EMB_DIM), jnp.float32),
        scratch_types=[
            pltpu.VMEM((PHASE_XR, RW), jnp.int32),
            [pltpu.VMEM((RW, EMB_DIM), jnp.float32) for _ in range(NBUF)],
            [pltpu.VMEM((NGS, EMB_DIM), jnp.float32) for _ in range(NBUF)],
            pltpu.VMEM((2, 16), jnp.float32),
            pltpu.VMEM((16,), jnp.int32),
            pltpu.VMEM((ROWS_PER_W, EMB_DIM), jnp.float32),
            pltpu.VMEM((8, EMB_DIM), jnp.float32),
            pltpu.VMEM((16,), jnp.int32),
            [pltpu.SemaphoreType.DMA for _ in range(NBUF)],
        ],
        compiler_params=pltpu.CompilerParams(
            needs_layout_passes=False, use_tc_tiling_on_sc=False
        ),
    )
    return k(xf, W)


# speedup vs baseline: 1.0422x; 1.0422x over previous
import functools

import jax
import jax.numpy as jnp
from jax import lax
from jax.experimental import pallas as pl
from jax.experimental.pallas import tpu as pltpu
from jax.experimental.pallas import tpu_sc as plsc

VOC_SIZE = 1000000
EMB_DIM = 32
BATCH = 16384
HIST_LEN = 200
WIN_A = 104
WIN_B = HIST_LEN - WIN_A

_info = plsc.get_sparse_core_info()
NC = _info.num_cores
NS = _info.num_subcores
NW = NC * NS
ROWS_PER_W = BATCH // NW
PHASE_ROWS = 256
NPHASE = ROWS_PER_W // PHASE_ROWS
NBUF = 4
NGROUP = PHASE_ROWS // NBUF


def _hsum16(vec, scratch_ref):
    lane = lax.iota(jnp.int32, 16)
    for sh in (8, 4, 2, 1):
        scratch_ref[...] = vec
        vec = vec + plsc.load_gather(scratch_ref, [lane ^ sh])
    return vec


def _body(x_hbm, w_hbm, out_hbm, idxa, idxb, bufs, out_v, w0_v, hs_v, sems):
    wid = lax.axis_index("s") * NC + lax.axis_index("c")

    pltpu.sync_copy(w_hbm.at[pl.ds(0, 8)], w0_v)
    w0a = w0_v[0, pl.ds(0, 16)]
    w0b = w0_v[0, pl.ds(16, 16)]

    zero = jnp.zeros((16,), jnp.float32)
    izero = jnp.zeros((16,), jnp.int32)
    ione = jnp.ones((16,), jnp.int32)
    lane = lax.iota(jnp.int32, 16)

    def fire(j, b):
        pltpu.async_copy(
            w_hbm.at[idxa.at[j]], bufs[b].at[pl.ds(0, WIN_A)], sems[b]
        )
        pltpu.async_copy(
            w_hbm.at[idxb.at[j]], bufs[b].at[pl.ds(WIN_A, WIN_B)], sems[b]
        )

    def drain(j, b):
        pltpu.make_async_copy(
            w_hbm.at[idxa.at[j]], bufs[b].at[pl.ds(0, WIN_A)], sems[b]
        ).wait()
        pltpu.make_async_copy(
            w_hbm.at[idxb.at[j]], bufs[b].at[pl.ds(WIN_A, WIN_B)], sems[b]
        ).wait()

    def phase(p, pcarry):
        row0 = wid * ROWS_PER_W + p * PHASE_ROWS
        pltpu.sync_copy(
            x_hbm.at[pl.ds(row0, PHASE_ROWS), pl.ds(0, WIN_A)], idxa
        )
        pltpu.sync_copy(
            x_hbm.at[pl.ds(row0, PHASE_ROWS), pl.ds(WIN_A, WIN_B)], idxb
        )
        for b in range(NBUF):
            fire(b, b)

        def group(g, carry):
            j0 = g * NBUF
            more = g < NGROUP - 1
            for b in range(NBUF):
                j = j0 + b
                drain(j, b)
                rv = bufs[b]
                t0 = zero
                t1 = zero
                for gi in range(HIST_LEN // 8):
                    s0 = zero
                    s1 = zero
                    for s in range(8 * gi, 8 * gi + 8):
                        s0 = s0 + rv[s, pl.ds(0, 16)]
                        s1 = s1 + rv[s, pl.ds(16, 16)]
                    t0 = t0 + s0
                    t1 = t1 + s1
                cnt = izero
                for o in range(0, 96, 16):
                    v = idxa[j, pl.ds(o, 16)]
                    cnt = cnt + jnp.where(v == 0, ione, izero)
                v = idxa[j, pl.ds(WIN_A - 16, 16)]
                cnt = cnt + jnp.where(
                    jnp.logical_and(v == 0, lane >= 16 - (WIN_A - 96)),
                    ione,
                    izero,
                )
                for o in range(0, WIN_B, 16):
                    v = idxb[j, pl.ds(o, 16)]
                    cnt = cnt + jnp.where(v == 0, ione, izero)

                @pl.when(more)
                def _(j=j, b=b):
                    fire(j + NBUF, b)

                nz = _hsum16(cnt, hs_v)
                nzf = nz.astype(jnp.float32)
                cntf = (HIST_LEN - nz).astype(jnp.float32)
                orow = p * PHASE_ROWS + j
                out_v[orow, pl.ds(0, 16)] = (t0 - nzf * w0a) / cntf
                out_v[orow, pl.ds(16, 16)] = (t1 - nzf * w0b) / cntf
            return carry

        lax.fori_loop(0, NGROUP, group, 0)
        return pcarry

    lax.fori_loop(0, NPHASE, phase, 0)

    pltpu.sync_copy(out_v, out_hbm.at[pl.ds(wid * ROWS_PER_W, ROWS_PER_W)])


@functools.partial(jax.jit, donate_argnums=())
def kernel(X, W):
    xi = X.astype(jnp.int32)
    mesh = plsc.VectorSubcoreMesh(core_axis_name="c", subcore_axis_name="s")
    k = pl.kernel(
        _body,
        mesh=mesh,
        out_type=jax.ShapeDtypeStruct((BATCH, EMB_DIM), jnp.float32),
        scratch_types=[
            pltpu.VMEM((PHASE_ROWS, WIN_A), jnp.int32),
            pltpu.VMEM((PHASE_ROWS, WIN_B), jnp.int32),
            [
                pltpu.VMEM((HIST_LEN, EMB_DIM), jnp.float32)
                for _ in range(NBUF)
            ],
            pltpu.VMEM((ROWS_PER_W, EMB_DIM), jnp.float32),
            pltpu.VMEM((8, EMB_DIM), jnp.float32),
            pltpu.VMEM((16,), jnp.int32),
            [pltpu.SemaphoreType.DMA for _ in range(NBUF)],
        ],
        compiler_params=pltpu.CompilerParams(
            needs_layout_passes=False, use_tc_tiling_on_sc=False
        ),
    )
    return k(xi, W)


# speedup vs baseline: 1.2664x; 1.2151x over previous
import functools

import jax
import jax.numpy as jnp
from jax import lax
from jax.experimental import pallas as pl
from jax.experimental.pallas import tpu as pltpu
from jax.experimental.pallas import tpu_sc as plsc

VOC_SIZE = 1000000
EMB_DIM = 32
BATCH = 16384
HIST_LEN = 200
HALF = 100

_info = plsc.get_sparse_core_info()
NC = _info.num_cores
NS = _info.num_subcores
NW = NC * NS
ROWS_PER_W = BATCH // NW
HALVES_PER_W = 2 * ROWS_PER_W
IDX_CHUNK = 512
NPHASE = HALVES_PER_W // IDX_CHUNK
NBUF = 8
NGROUP = IDX_CHUNK // NBUF


def _count_zeros(idx_ref, r):
    lane = lax.iota(jnp.int32, 16)
    one = jnp.ones((16,), jnp.int32)
    nil = jnp.zeros((16,), jnp.int32)
    cnt = nil
    for o in (0, 16, 32, 48, 64, 80):
        v = idx_ref[r, pl.ds(o, 16)]
        cnt = cnt + jnp.where(v == 0, one, nil)
    v = idx_ref[r, pl.ds(84, 16)]
    cnt = cnt + jnp.where(jnp.logical_and(v == 0, lane >= 12), one, nil)
    return cnt


def _hsum16(vec, scratch_ref):
    lane = lax.iota(jnp.int32, 16)
    for sh in (8, 4, 2, 1):
        scratch_ref[...] = vec
        vec = vec + plsc.load_gather(scratch_ref, [lane ^ sh])
    return vec


def _body(x2_hbm, w_hbm, out_hbm, idx_v, bufs, out_v, w0_v, hs_v, sems):
    wid = lax.axis_index("s") * NC + lax.axis_index("c")
    base_h = wid * HALVES_PER_W

    pltpu.sync_copy(w_hbm.at[pl.ds(0, 8)], w0_v)
    w0a = w0_v[0, pl.ds(0, 16)]
    w0b = w0_v[0, pl.ds(16, 16)]

    zero = jnp.zeros((16,), jnp.float32)

    def fire(h, b):
        pltpu.async_copy(w_hbm.at[idx_v.at[h]], bufs[b], sems[b])

    def drain(h, b):
        pltpu.make_async_copy(w_hbm.at[idx_v.at[h]], bufs[b], sems[b]).wait()

    for p in range(NPHASE):
        pltpu.sync_copy(
            x2_hbm.at[pl.ds(base_h + p * IDX_CHUNK, IDX_CHUNK)], idx_v
        )
        for b in range(NBUF):
            fire(b, b)

        def group(g, carry, p=p):
            h0 = g * NBUF
            more = g < NGROUP - 1
            for pairb in range(NBUF // 2):
                acc0 = zero
                acc1 = zero
                nz = None
                for b in (2 * pairb, 2 * pairb + 1):
                    h = h0 + b
                    drain(h, b)
                    rv = bufs[b]
                    for i in range(HALF):
                        acc0 = acc0 + rv[i, pl.ds(0, 16)]
                        acc1 = acc1 + rv[i, pl.ds(16, 16)]
                    zc = _count_zeros(idx_v, h)
                    nz = zc if nz is None else nz + zc

                    @pl.when(more)
                    def _(h=h, b=b):
                        fire(h + NBUF, b)

                nz = _hsum16(nz, hs_v)
                nzf = nz.astype(jnp.float32)
                cntf = (HIST_LEN - nz).astype(jnp.float32)
                orow = p * (IDX_CHUNK // 2) + (h0 // 2) + pairb
                out_v[orow, pl.ds(0, 16)] = (acc0 - nzf * w0a) / cntf
                out_v[orow, pl.ds(16, 16)] = (acc1 - nzf * w0b) / cntf
            return carry

        lax.fori_loop(0, NGROUP, group, 0)

    pltpu.sync_copy(out_v, out_hbm.at[pl.ds(wid * ROWS_PER_W, ROWS_PER_W)])


@functools.partial(jax.jit, donate_argnums=())
def kernel(X, W):
    X2 = X.astype(jnp.int32).reshape(BATCH * 2, HALF)
    mesh = plsc.VectorSubcoreMesh(core_axis_name="c", subcore_axis_name="s")
    k = pl.kernel(
        _body,
        mesh=mesh,
        out_type=jax.ShapeDtypeStruct((BATCH, EMB_DIM), jnp.float32),
        scratch_types=[
            pltpu.VMEM((IDX_CHUNK, HALF), jnp.int32),
            [pltpu.VMEM((HALF, EMB_DIM), jnp.float32) for _ in range(NBUF)],
            pltpu.VMEM((ROWS_PER_W, EMB_DIM), jnp.float32),
            pltpu.VMEM((8, EMB_DIM), jnp.float32),
            pltpu.VMEM((16,), jnp.int32),
            [pltpu.SemaphoreType.DMA for _ in range(NBUF)],
        ],
        compiler_params=pltpu.CompilerParams(
            needs_layout_passes=False, use_tc_tiling_on_sc=False
        ),
    )
    return k(X2, W)
